# baseline (speedup 1.0000x reference)
import jax
import jax.numpy as jnp
from jax import lax
from jax.experimental import pallas as pl
from jax.experimental.pallas import tpu as pltpu
from jax.experimental.pallas import tpu_sc as plsc

N = 10000
E = 320000
D = 128
WF = 144
NB = 10240
NC = 2
NS = 16
L = 16
EPT = 20480
ESLOTS = NS * EPT
CHW = 80
BLK = 3
RING = 4
GRP = RING * BLK * CHW
CAP = ((EPT + GRP - 1) // GRP + 1) * GRP
CAPCH = CAP // CHW
STRIPE = NB // NS
PACKB = 15

_f32 = jnp.float32
_i32 = jnp.int32


def _build_body(x_ref, w_ref, avs_ref, avd_ref, g_ref, h_ref, as_ref, ad_ref):
    h = jnp.dot(x_ref[...], w_ref[...], preferred_element_type=_f32)
    a_s = jnp.dot(h, avs_ref[...], preferred_element_type=_f32)
    a_d = jnp.dot(h, avd_ref[...], preferred_element_type=_f32)
    e1 = jnp.exp(a_s)
    e2 = jnp.exp(0.2 * a_s)
    zpad = jnp.zeros((h.shape[0], WF - D - 1), _f32)
    g0 = jnp.concatenate([e1 * h, e1, zpad], axis=1)
    g1 = jnp.concatenate([e2 * h, e2, zpad], axis=1)
    g_ref[...] = jnp.stack([g0, g1], axis=0)
    h_ref[...] = h
    as_ref[...] = a_s
    ad_ref[...] = a_d


def _build_call(x_pad, W, av_s, av_d):
    BN = 1280
    grid = NB // BN
    return pl.pallas_call(
        _build_body,
        grid=(grid,),
        in_specs=[
            pl.BlockSpec((BN, D), lambda i: (i, 0)),
            pl.BlockSpec((D, D), lambda i: (0, 0)),
            pl.BlockSpec((D, 1), lambda i: (0, 0)),
            pl.BlockSpec((D, 1), lambda i: (0, 0)),
        ],
        out_specs=[
            pl.BlockSpec((2, BN, WF), lambda i: (0, i, 0)),
            pl.BlockSpec((BN, D), lambda i: (i, 0)),
            pl.BlockSpec((BN, 1), lambda i: (i, 0)),
            pl.BlockSpec((BN, 1), lambda i: (i, 0)),
        ],
        out_shape=[
            jax.ShapeDtypeStruct((2, NB, WF), _f32),
            jax.ShapeDtypeStruct((NB, D), _f32),
            jax.ShapeDtypeStruct((NB, 1), _f32),
            jax.ShapeDtypeStruct((NB, 1), _f32),
        ],
    )(x_pad, W, av_s, av_d)


def _sign_body(as_hbm, ad_hbm, src_hbm, dst_hbm, lst_hbm, cnt_hbm,
               asv, adv, srcv, dstv, la, lb, cntv):
    s = lax.axis_index("s")

    pltpu.sync_copy(as_hbm, asv)
    pltpu.sync_copy(ad_hbm, adv)
    pltpu.sync_copy(src_hbm.at[s], srcv)
    pltpu.sync_copy(dst_hbm.at[s], dstv)

    nbvec = jnp.full((L,), NB, _i32)
    zvec = jnp.zeros((L,), _i32)

    def step(j, carry):
        offa, offb = carry
        s16 = srcv[pl.ds(j * L, L)]
        d16 = dstv[pl.ds(j * L, L)]
        z = plsc.load_gather(asv, [s16]) + plsc.load_gather(adv, [d16])
        neg = z < 0.0
        pos = jnp.logical_not(neg)
        gsrc = s16 + jnp.where(neg, nbvec, zvec)
        packed = gsrc | lax.shift_left(d16, PACKB)
        plsc.store_compressed(la.at[pl.ds(offa, L)], packed, mask=pos)
        plsc.store_compressed(lb.at[pl.ds(offb, L)], packed, mask=neg)
        pa = jnp.max(plsc.all_reduce_population_count(pos))
        pb = jnp.max(plsc.all_reduce_population_count(neg))
        return offa + pa, offb + pb

    offa, offb = lax.fori_loop(0, EPT // L, step,
                               (jnp.int32(0), jnp.int32(0)))

    for cls, lref, off in ((0, la, offa), (1, lb, offb)):
        tgt = ((off + GRP - 1) // GRP) * GRP
        tgt = jnp.maximum(tgt, GRP)
        trash16 = jnp.full((L,), (cls * NB + N) | (N << PACKB), _i32)

        def pad(i, carry):
            lref[pl.ds(off + i * L, L)] = trash16
            return carry

        lax.fori_loop(0, (tgt - off + L - 1) // L, pad, 0)
        cntv[...] = jnp.broadcast_to(tgt, (L,)).astype(_i32)
        pltpu.sync_copy(cntv, cnt_hbm.at[cls, s])
        pltpu.sync_copy(lref, lst_hbm.at[cls, s])


def _sign_call():
  return pl.kernel(
    _sign_body,
    out_type=[
        jax.ShapeDtypeStruct((NC, NS, CAP + L), _i32),
        jax.ShapeDtypeStruct((NC, NS, L), _i32),
    ],
    mesh=plsc.VectorSubcoreMesh(core_axis_name="c", subcore_axis_name="s"),
    compiler_params=pltpu.CompilerParams(needs_layout_passes=False,
                                         use_tc_tiling_on_sc=False),
    scratch_types=[
        pltpu.VMEM((NB,), _f32),
        pltpu.VMEM((NB,), _f32),
        pltpu.VMEM((EPT,), _i32),
        pltpu.VMEM((EPT,), _i32),
        pltpu.VMEM((CAP + L,), _i32),
        pltpu.VMEM((CAP + L,), _i32),
        pltpu.VMEM((L,), _i32),
    ],
  )


def _gs_body(g_hbm, lst_hbm, cnt_hbm, out_hbm, pidx, sidx, didx, rows, acc,
             cntv, semg0, semg1, semg2, sems0, sems1, sems2,
             si0, si1, si2, si3):
    c = lax.axis_index("c")
    s = lax.axis_index("s")
    semg = (semg0, semg1, semg2)
    sems = (sems0, sems1, sems2)
    semi = (si0, si1, si2, si3)

    zero16 = jnp.zeros((L,), _f32)

    def zrow(r, carry):
        for q in range(WF // L):
            rows[0, r, pl.ds(q * L, L)] = zero16
        return carry

    lax.fori_loop(0, CHW, zrow, 0)
    base = s * STRIPE
    for k in range(STRIPE // CHW):
        pltpu.sync_copy(rows.at[0], acc.at[pl.ds(base + k * CHW, CHW)])
    plsc.subcore_barrier()

    pltpu.sync_copy(cnt_hbm.at[c, s], cntv)
    cnt = jnp.max(cntv[...])
    nblk = cnt // (BLK * CHW)
    ngrp = cnt // GRP

    def stage(k, sl):
        pltpu.async_copy(lst_hbm.at[c, s, pl.ds(k * BLK, BLK)],
                         pidx.at[sl], semi[sl])

    def stage_wait_unpack(k, sl):
        pltpu.make_async_copy(lst_hbm.at[c, s, pl.ds(k * BLK, BLK)],
                              pidx.at[sl], semi[sl]).wait()
        for ch in range(BLK):
            for q in range(CHW // L):
                p = pidx[sl, ch, pl.ds(q * L, L)]
                sidx[sl, ch, pl.ds(q * L, L)] = p & ((1 << PACKB) - 1)
                didx[sl, ch, pl.ds(q * L, L)] = lax.shift_right_logical(
                    p, PACKB)

    def drain_scatter(b):
        pltpu.make_async_copy(rows.at[b], acc.at[didx.at[0, 0]],
                              sems[b]).wait()

    def drain_gather(b):
        pltpu.make_async_copy(g_hbm.at[sidx.at[0, 0]], rows.at[b],
                              semg[b]).wait()

    def block(k, sl, psl, first):
        stage_wait_unpack(k, sl)
        for m in range(BLK):
            b = m
            pm = (m - 1) % BLK
            if not first:
                drain_scatter(b)
            pltpu.async_copy(g_hbm.at[sidx.at[sl, m]], rows.at[b], semg[b])
            if not (first and m == 0):
                drain_gather(pm)
                if m > 0:
                    pltpu.async_copy(rows.at[pm], acc.at[didx.at[sl, m - 1]],
                                     sems[pm], add=True)
                else:
                    pltpu.async_copy(rows.at[pm],
                                     acc.at[didx.at[psl, BLK - 1]],
                                     sems[pm], add=True)

    for kk in range(RING):
        stage(kk, kk)
    block(0, 0, None, True)
    block(1, 1, 0, False)

    @pl.when(4 < nblk)
    def _():
        stage(4, 0)

    block(2, 2, 1, False)

    @pl.when(5 < nblk)
    def _():
        stage(5, 1)

    block(3, 3, 2, False)

    def blocks(g, carry):
        for b4 in range(RING):
            k = RING * g + b4

            @pl.when(k + 2 < nblk)
            def _():
                stage(k + 2, (b4 + 2) % RING)

            block(k, b4, (b4 - 1) % RING, False)
        return carry

    lax.fori_loop(1, ngrp, blocks, 0)

    drain_gather(BLK - 1)
    pltpu.async_copy(rows.at[BLK - 1], acc.at[didx.at[RING - 1, BLK - 1]],
                     sems[BLK - 1], add=True)
    for b in range(BLK):
        drain_scatter(b)
    plsc.subcore_barrier()

    pltpu.sync_copy(acc.at[pl.ds(base, STRIPE)],
                    out_hbm.at[c, pl.ds(base, STRIPE)])


def _gs_call():
  return pl.kernel(
    _gs_body,
    out_type=jax.ShapeDtypeStruct((NC, NB, WF), _f32),
    mesh=plsc.VectorSubcoreMesh(core_axis_name="c", subcore_axis_name="s"),
    compiler_params=pltpu.CompilerParams(needs_layout_passes=False,
                                         use_tc_tiling_on_sc=False),
    scratch_types=[
        pltpu.VMEM((RING, BLK, CHW), _i32),
        pltpu.VMEM((RING, BLK, CHW), _i32),
        pltpu.VMEM((RING, BLK, CHW), _i32),
        pltpu.VMEM((BLK, CHW, WF), _f32),
        pltpu.VMEM_SHARED((NB, WF), _f32),
        pltpu.VMEM((L,), _i32),
    ] + [pltpu.SemaphoreType.DMA] * 10,
  )


def _sc_edge_pass(G2, a_s, a_d, srcp, dstp):
    lists, counts = _sign_call()(a_s, a_d, srcp, dstp)
    lists = lists[:, :, :CAP].reshape(NC, NS, CAPCH, CHW)
    return _gs_call()(G2, lists, counts)


def _combine_body(acc_ref, as_ref, ad_ref, h_ref, b_ref, out_ref):
    a_s = as_ref[...]
    a_d = ad_ref[...]
    h = h_ref[...]
    e1 = jnp.exp(a_d)
    e2 = jnp.exp(0.2 * a_d)
    agg = e1 * acc_ref[0] + e2 * acc_ref[1]
    z = a_s + a_d
    ws = jnp.exp(jnp.where(z >= 0.0, z, 0.2 * z))
    num = agg[:, :D] + ws * h
    den = agg[:, D:D + 1] + ws
    out_ref[...] = num / den + b_ref[...]


def _combine_call(acc, a_s, a_d, h, bias):
    BN = 1000
    grid = N // BN
    return pl.pallas_call(
        _combine_body,
        grid=(grid,),
        in_specs=[
            pl.BlockSpec((2, BN, WF), lambda i: (0, i, 0)),
            pl.BlockSpec((BN, 1), lambda i: (i, 0)),
            pl.BlockSpec((BN, 1), lambda i: (i, 0)),
            pl.BlockSpec((BN, D), lambda i: (i, 0)),
            pl.BlockSpec((1, D), lambda i: (0, 0)),
        ],
        out_specs=pl.BlockSpec((BN, D), lambda i: (i, 0)),
        out_shape=jax.ShapeDtypeStruct((N, D), _f32),
    )(acc, a_s, a_d, h, bias)


def _mlp1_body(x_ref, c0_ref, c1_ref, c2_ref, w1_ref, b1_ref, h1_ref, st_ref):
    h1 = (jnp.dot(x_ref[...], w1_ref[0], preferred_element_type=_f32)
          + jnp.dot(c0_ref[...], w1_ref[1], preferred_element_type=_f32)
          + jnp.dot(c1_ref[...], w1_ref[2], preferred_element_type=_f32)
          + jnp.dot(c2_ref[...], w1_ref[3], preferred_element_type=_f32)
          + b1_ref[...])
    h1_ref[...] = h1
    part = jnp.concatenate([jnp.sum(h1, axis=0, keepdims=True),
                            jnp.sum(h1 * h1, axis=0, keepdims=True)], axis=0)

    @pl.when(pl.program_id(0) == 0)
    def _():
        st_ref[...] = jnp.zeros_like(st_ref)

    st_ref[...] += part


def _mlp1_call(x, c0, c1, c2, w1_4, b1):
    BN = 1000
    grid = N // BN
    return pl.pallas_call(
        _mlp1_body,
        grid=(grid,),
        in_specs=[
            pl.BlockSpec((BN, D), lambda i: (i, 0)),
            pl.BlockSpec((BN, D), lambda i: (i, 0)),
            pl.BlockSpec((BN, D), lambda i: (i, 0)),
            pl.BlockSpec((BN, D), lambda i: (i, 0)),
            pl.BlockSpec((4, D, D), lambda i: (0, 0, 0)),
            pl.BlockSpec((1, D), lambda i: (0, 0)),
        ],
        out_specs=[
            pl.BlockSpec((BN, D), lambda i: (i, 0)),
            pl.BlockSpec((2, D), lambda i: (0, 0)),
        ],
        out_shape=[
            jax.ShapeDtypeStruct((N, D), _f32),
            jax.ShapeDtypeStruct((2, D), _f32),
        ],
    )(x, c0, c1, c2, w1_4, b1)


def _mlp2_body(h1_ref, st_ref, g_ref, be_ref, w2_ref, b2_ref, out_ref):
    mean = st_ref[0:1] * (1.0 / N)
    var = st_ref[1:2] * (1.0 / N) - mean * mean
    xn = (h1_ref[...] - mean) * lax.rsqrt(var + 1e-5) * g_ref[...] + be_ref[...]
    xn = jnp.maximum(xn, 0.0)
    out_ref[...] = jnp.dot(xn, w2_ref[...],
                           preferred_element_type=_f32) + b2_ref[...]


def _mlp2_call(h1, stats, gamma, beta, w2, b2):
    BN = 1000
    grid = N // BN
    return pl.pallas_call(
        _mlp2_body,
        grid=(grid,),
        in_specs=[
            pl.BlockSpec((BN, D), lambda i: (i, 0)),
            pl.BlockSpec((2, D), lambda i: (0, 0)),
            pl.BlockSpec((1, D), lambda i: (0, 0)),
            pl.BlockSpec((1, D), lambda i: (0, 0)),
            pl.BlockSpec((D, D), lambda i: (0, 0)),
            pl.BlockSpec((1, D), lambda i: (0, 0)),
        ],
        out_specs=pl.BlockSpec((BN, D), lambda i: (i, 0)),
        out_shape=jax.ShapeDtypeStruct((N, D), _f32),
    )(h1, stats, gamma, beta, w2, b2)


def kernel(x, edge_index, t, batch_ids,
           W0, att_src0, att_dst0, bias0,
           W1, att_src1, att_dst1, bias1,
           W2, att_src2, att_dst2, bias2,
           mlp_W1, mlp_b1, mlp_gamma, mlp_beta, mlp_W2, mlp_b2):
    del t, batch_ids
    pad_e = ESLOTS - E
    srcp = jnp.concatenate(
        [edge_index[0], jnp.full((pad_e,), N, _i32)]).reshape(NS, EPT)
    dstp = jnp.concatenate(
        [edge_index[1], jnp.full((pad_e,), N, _i32)]).reshape(NS, EPT)

    feat = jnp.pad(x, ((0, NB - N), (0, 0)))
    convs = []
    params = [(W0, att_src0, att_dst0, bias0),
              (W1, att_src1, att_dst1, bias1),
              (W2, att_src2, att_dst2, bias2)]
    for (W, a_s_w, a_d_w, b) in params:
        G2, h, a_s, a_d = _build_call(
            feat, W, a_s_w.reshape(D, 1), a_d_w.reshape(D, 1))
        acc = _sc_edge_pass(
            G2.reshape(2 * NB, WF), a_s.reshape(NB), a_d.reshape(NB),
            srcp, dstp)
        conv = _combine_call(acc, a_s, a_d, h, b.reshape(1, D))
        convs.append(conv)
        feat = jnp.pad(conv, ((0, NB - N), (0, 0)))

    h1, stats = _mlp1_call(x, convs[0], convs[1], convs[2],
                           mlp_W1.reshape(4, D, D), mlp_b1.reshape(1, D))
    return _mlp2_call(h1, stats, mlp_gamma.reshape(1, D),
                      mlp_beta.reshape(1, D), mlp_W2, mlp_b2.reshape(1, D))


# speedup vs baseline: 1.9351x; 1.9351x over previous
import functools

import jax
import jax.numpy as jnp
from jax import lax
from jax.experimental import pallas as pl
from jax.experimental.pallas import tpu as pltpu
from jax.experimental.pallas import tpu_sc as plsc

N = 10000
E = 320000
D = 128
HALF = D // 2
WCOL = 72
NB = 10240
NC = 2
NS = 16
L = 16
CHW = 128
NCH = 160
EPT = NCH * CHW
ESLOTS = NS * EPT
STRIPE = 2 * NB // NS

_f32 = jnp.float32
_i32 = jnp.int32


def _build_body(x_ref, w_ref, avs_ref, avd_ref, g_ref, h_ref, as_ref, ad_ref):
    h = jnp.dot(x_ref[...], w_ref[...], preferred_element_type=_f32)
    a_s = jnp.dot(h, avs_ref[...], preferred_element_type=_f32)
    a_d = jnp.dot(h, avd_ref[...], preferred_element_type=_f32)
    e1 = jnp.exp(a_s)
    e2 = jnp.exp(0.2 * a_s)
    zpad = jnp.zeros((h.shape[0], WCOL - HALF - 1), _f32)
    lo = h[:, :HALF]
    hi = h[:, HALF:]
    g0 = jnp.concatenate([e1 * lo, e1, zpad], axis=1)
    g1 = jnp.concatenate([e2 * lo, e2, zpad], axis=1)
    g2 = jnp.concatenate([e1 * hi, e1, zpad], axis=1)
    g3 = jnp.concatenate([e2 * hi, e2, zpad], axis=1)
    g_ref[...] = jnp.stack([g0, g1, g2, g3], axis=0)
    h_ref[...] = h
    as_ref[...] = a_s
    ad_ref[...] = a_d


def _build_call(x_pad, W, av_s, av_d):
    BN = 1280
    grid = NB // BN
    return pl.pallas_call(
        _build_body,
        grid=(grid,),
        in_specs=[
            pl.BlockSpec((BN, D), lambda i: (i, 0)),
            pl.BlockSpec((D, D), lambda i: (0, 0)),
            pl.BlockSpec((D, 1), lambda i: (0, 0)),
            pl.BlockSpec((D, 1), lambda i: (0, 0)),
        ],
        out_specs=[
            pl.BlockSpec((4, BN, WCOL), lambda i: (0, i, 0)),
            pl.BlockSpec((BN, D), lambda i: (i, 0)),
            pl.BlockSpec((BN, 1), lambda i: (i, 0)),
            pl.BlockSpec((BN, 1), lambda i: (i, 0)),
        ],
        out_shape=[
            jax.ShapeDtypeStruct((4, NB, WCOL), _f32),
            jax.ShapeDtypeStruct((NB, D), _f32),
            jax.ShapeDtypeStruct((NB, 1), _f32),
            jax.ShapeDtypeStruct((NB, 1), _f32),
        ],
    )(x_pad, W, av_s, av_d)


def _sign_body(as_hbm, ad_hbm, src_hbm, dst_hbm, gsrc_hbm, gdst_hbm,
               asv, adv, srcv, dstv):
    c = lax.axis_index("c")
    s = lax.axis_index("s")

    pltpu.sync_copy(as_hbm, asv)
    pltpu.sync_copy(ad_hbm, adv)
    half = NCH // NC
    pltpu.sync_copy(src_hbm.at[s, pl.ds(c * half, half)], srcv)
    pltpu.sync_copy(dst_hbm.at[s, pl.ds(c * half, half)], dstv)

    nbvec = jnp.full((L,), NB, _i32)
    zvec = jnp.zeros((L,), _i32)
    c1vec = jnp.full((L,), 2 * NB, _i32)

    def sgn(j, carry):
        for i in range(CHW // L):
            s16 = srcv[j, pl.ds(i * L, L)]
            d16 = dstv[j, pl.ds(i * L, L)]
            z = plsc.load_gather(asv, [s16]) + plsc.load_gather(adv, [d16])
            offs = jnp.where(z < 0.0, nbvec, zvec)
            srcv[j, pl.ds(i * L, L)] = s16 + offs
            dstv[j, pl.ds(i * L, L)] = d16 + offs
        return carry

    lax.fori_loop(0, half, sgn, 0)
    pltpu.sync_copy(srcv, gsrc_hbm.at[0, s, pl.ds(c * half, half)])
    pltpu.sync_copy(dstv, gdst_hbm.at[s, pl.ds(c * half, half)])

    def sft(j, carry):
        for i in range(CHW // L):
            srcv[j, pl.ds(i * L, L)] += c1vec
        return carry

    lax.fori_loop(0, half, sft, 0)
    pltpu.sync_copy(srcv, gsrc_hbm.at[1, s, pl.ds(c * half, half)])


def _sign_call():
  return pl.kernel(
    _sign_body,
    out_type=[
        jax.ShapeDtypeStruct((NC, NS, NCH, CHW), _i32),
        jax.ShapeDtypeStruct((NS, NCH, CHW), _i32),
    ],
    mesh=plsc.VectorSubcoreMesh(core_axis_name="c", subcore_axis_name="s"),
    compiler_params=pltpu.CompilerParams(needs_layout_passes=False,
                                         use_tc_tiling_on_sc=False),
    scratch_types=[
        pltpu.VMEM((NB,), _f32),
        pltpu.VMEM((NB,), _f32),
        pltpu.VMEM((NCH // NC, CHW), _i32),
        pltpu.VMEM((NCH // NC, CHW), _i32),
    ],
  )


BLK = 4
RING = 4
NBLK = NCH // BLK


def _gs_body(g_hbm, gsrc_hbm, gdst_hbm, out_hbm, sidx, didx, rows, acc,
             semg0, semg1, sems0, sems1, si0, si1, si2, si3):
    c = lax.axis_index("c")
    s = lax.axis_index("s")
    semg = (semg0, semg1)
    sems = (sems0, sems1)
    semi = (si0, si1, si2, si3)

    zero16 = jnp.zeros((L,), _f32)

    def zrow(r, carry):
        for q in (0, 16, 32, 48, WCOL - L):
            rows[0, r, pl.ds(q, L)] = zero16
        return carry

    lax.fori_loop(0, CHW, zrow, 0)
    base = s * STRIPE
    for k in range(STRIPE // CHW):
        pltpu.sync_copy(rows.at[0], acc.at[pl.ds(base + k * CHW, CHW)])
    plsc.subcore_barrier()

    def stage(k, sl):
        pltpu.async_copy(gsrc_hbm.at[c, s, pl.ds(k * BLK, BLK)],
                         sidx.at[sl], semi[sl])
        pltpu.async_copy(gdst_hbm.at[s, pl.ds(k * BLK, BLK)],
                         didx.at[sl], semi[sl])

    def stage_wait(k, sl):
        pltpu.make_async_copy(gsrc_hbm.at[c, s, pl.ds(k * BLK, BLK)],
                              sidx.at[sl], semi[sl]).wait()
        pltpu.make_async_copy(gdst_hbm.at[s, pl.ds(k * BLK, BLK)],
                              didx.at[sl], semi[sl]).wait()

    def drain_scatter(b):
        pltpu.make_async_copy(rows.at[b], acc.at[didx.at[0, 0]],
                              sems[b]).wait()

    def drain_gather(b):
        pltpu.make_async_copy(g_hbm.at[sidx.at[0, 0]], rows.at[b],
                              semg[b]).wait()

    def block(k, sl, psl, first):
        stage_wait(k, sl)
        for m in range(BLK):
            b = m % 2
            nb = 1 - b
            if not (first and m < 2):
                drain_scatter(b)
            pltpu.async_copy(g_hbm.at[sidx.at[sl, m]], rows.at[b], semg[b])
            if not (first and m == 0):
                drain_gather(nb)
                if m > 0:
                    pltpu.async_copy(rows.at[nb], acc.at[didx.at[sl, m - 1]],
                                     sems[nb], add=True)
                else:
                    pltpu.async_copy(rows.at[nb],
                                     acc.at[didx.at[psl, BLK - 1]],
                                     sems[nb], add=True)

    for kk in range(RING):
        stage(kk, kk)
    block(0, 0, None, True)
    block(1, 1, 0, False)
    stage(4, 0)
    block(2, 2, 1, False)
    stage(5, 1)
    block(3, 3, 2, False)

    def blocks(g, carry):
        for b4 in range(RING):
            k = RING * g + b4

            @pl.when(k + 2 < NBLK)
            def _():
                stage(k + 2, (b4 + 2) % RING)

            block(k, b4, (b4 - 1) % RING, False)
        return carry

    lax.fori_loop(1, NBLK // RING, blocks, 0)

    drain_gather(1)
    pltpu.async_copy(rows.at[1], acc.at[didx.at[RING - 1, BLK - 1]],
                     sems[1], add=True)
    drain_scatter(0)
    drain_scatter(1)
    plsc.subcore_barrier()

    pltpu.sync_copy(acc.at[pl.ds(base, STRIPE)],
                    out_hbm.at[c, pl.ds(base, STRIPE)])


def _gs_call():
  return pl.kernel(
    _gs_body,
    out_type=jax.ShapeDtypeStruct((NC, 2 * NB, WCOL), _f32),
    mesh=plsc.VectorSubcoreMesh(core_axis_name="c", subcore_axis_name="s"),
    compiler_params=pltpu.CompilerParams(needs_layout_passes=False,
                                         use_tc_tiling_on_sc=False),
    scratch_types=[
        pltpu.VMEM((RING, BLK, CHW), _i32),
        pltpu.VMEM((RING, BLK, CHW), _i32),
        pltpu.VMEM((2, CHW, WCOL), _f32),
        pltpu.VMEM_SHARED((2 * NB, WCOL), _f32),
        pltpu.SemaphoreType.DMA,
        pltpu.SemaphoreType.DMA,
        pltpu.SemaphoreType.DMA,
        pltpu.SemaphoreType.DMA,
        pltpu.SemaphoreType.DMA,
        pltpu.SemaphoreType.DMA,
        pltpu.SemaphoreType.DMA,
        pltpu.SemaphoreType.DMA,
    ],
  )


def _sc_edge_pass(G, a_s, a_d, srcp, dstp):
    gsrc, gdst = _sign_call()(a_s, a_d, srcp, dstp)
    return _gs_call()(G, gsrc, gdst)


def _combine_body(acc_ref, as_ref, ad_ref, h_ref, b_ref, out_ref):
    a_s = as_ref[...]
    a_d = ad_ref[...]
    h = h_ref[...]
    e1 = jnp.exp(a_d)
    e2 = jnp.exp(0.2 * a_d)
    lo = e1 * acc_ref[0, 0] + e2 * acc_ref[0, 1]
    hi = e1 * acc_ref[1, 0] + e2 * acc_ref[1, 1]
    z = a_s + a_d
    ws = jnp.exp(jnp.where(z >= 0.0, z, 0.2 * z))
    num = jnp.concatenate([lo[:, :HALF], hi[:, :HALF]], axis=1) + ws * h
    den = lo[:, HALF:HALF + 1] + ws
    out_ref[...] = num / den + b_ref[...]


def _combine_call(acc4, a_s, a_d, h, bias):
    BN = 1000
    grid = N // BN
    return pl.pallas_call(
        _combine_body,
        grid=(grid,),
        in_specs=[
            pl.BlockSpec((2, 2, BN, WCOL), lambda i: (0, 0, i, 0)),
            pl.BlockSpec((BN, 1), lambda i: (i, 0)),
            pl.BlockSpec((BN, 1), lambda i: (i, 0)),
            pl.BlockSpec((BN, D), lambda i: (i, 0)),
            pl.BlockSpec((1, D), lambda i: (0, 0)),
        ],
        out_specs=pl.BlockSpec((BN, D), lambda i: (i, 0)),
        out_shape=jax.ShapeDtypeStruct((N, D), _f32),
    )(acc4, a_s, a_d, h, bias)


def _mlp1_body(x_ref, c0_ref, c1_ref, c2_ref, w1_ref, b1_ref, h1_ref, st_ref):
    h1 = (jnp.dot(x_ref[...], w1_ref[0], preferred_element_type=_f32)
          + jnp.dot(c0_ref[...], w1_ref[1], preferred_element_type=_f32)
          + jnp.dot(c1_ref[...], w1_ref[2], preferred_element_type=_f32)
          + jnp.dot(c2_ref[...], w1_ref[3], preferred_element_type=_f32)
          + b1_ref[...])
    h1_ref[...] = h1
    part = jnp.concatenate([jnp.sum(h1, axis=0, keepdims=True),
                            jnp.sum(h1 * h1, axis=0, keepdims=True)], axis=0)

    @pl.when(pl.program_id(0) == 0)
    def _():
        st_ref[...] = jnp.zeros_like(st_ref)

    st_ref[...] += part


def _mlp1_call(x, c0, c1, c2, w1_4, b1):
    BN = 1000
    grid = N // BN
    return pl.pallas_call(
        _mlp1_body,
        grid=(grid,),
        in_specs=[
            pl.BlockSpec((BN, D), lambda i: (i, 0)),
            pl.BlockSpec((BN, D), lambda i: (i, 0)),
            pl.BlockSpec((BN, D), lambda i: (i, 0)),
            pl.BlockSpec((BN, D), lambda i: (i, 0)),
            pl.BlockSpec((4, D, D), lambda i: (0, 0, 0)),
            pl.BlockSpec((1, D), lambda i: (0, 0)),
        ],
        out_specs=[
            pl.BlockSpec((BN, D), lambda i: (i, 0)),
            pl.BlockSpec((2, D), lambda i: (0, 0)),
        ],
        out_shape=[
            jax.ShapeDtypeStruct((N, D), _f32),
            jax.ShapeDtypeStruct((2, D), _f32),
        ],
    )(x, c0, c1, c2, w1_4, b1)


def _mlp2_body(h1_ref, st_ref, g_ref, be_ref, w2_ref, b2_ref, out_ref):
    mean = st_ref[0:1] * (1.0 / N)
    var = st_ref[1:2] * (1.0 / N) - mean * mean
    xn = (h1_ref[...] - mean) * lax.rsqrt(var + 1e-5) * g_ref[...] + be_ref[...]
    xn = jnp.maximum(xn, 0.0)
    out_ref[...] = jnp.dot(xn, w2_ref[...], preferred_element_type=_f32) + b2_ref[...]


def _mlp2_call(h1, stats, gamma, beta, w2, b2):
    BN = 1000
    grid = N // BN
    return pl.pallas_call(
        _mlp2_body,
        grid=(grid,),
        in_specs=[
            pl.BlockSpec((BN, D), lambda i: (i, 0)),
            pl.BlockSpec((2, D), lambda i: (0, 0)),
            pl.BlockSpec((1, D), lambda i: (0, 0)),
            pl.BlockSpec((1, D), lambda i: (0, 0)),
            pl.BlockSpec((D, D), lambda i: (0, 0)),
            pl.BlockSpec((1, D), lambda i: (0, 0)),
        ],
        out_specs=pl.BlockSpec((BN, D), lambda i: (i, 0)),
        out_shape=jax.ShapeDtypeStruct((N, D), _f32),
    )(h1, stats, gamma, beta, w2, b2)


def kernel(x, edge_index, t, batch_ids,
           W0, att_src0, att_dst0, bias0,
           W1, att_src1, att_dst1, bias1,
           W2, att_src2, att_dst2, bias2,
           mlp_W1, mlp_b1, mlp_gamma, mlp_beta, mlp_W2, mlp_b2):
    del t, batch_ids
    pad_e = ESLOTS - E
    srcp = jnp.concatenate(
        [edge_index[0], jnp.full((pad_e,), N, _i32)]).reshape(NS, NCH, CHW)
    dstp = jnp.concatenate(
        [edge_index[1], jnp.full((pad_e,), N, _i32)]).reshape(NS, NCH, CHW)

    feat = jnp.pad(x, ((0, NB - N), (0, 0)))
    convs = []
    params = [(W0, att_src0, att_dst0, bias0),
              (W1, att_src1, att_dst1, bias1),
              (W2, att_src2, att_dst2, bias2)]
    for (W, a_s_w, a_d_w, b) in params:
        G4, h, a_s, a_d = _build_call(
            feat, W, a_s_w.reshape(D, 1), a_d_w.reshape(D, 1))
        acc = _sc_edge_pass(
            G4.reshape(4 * NB, WCOL), a_s.reshape(NB), a_d.reshape(NB),
            srcp, dstp)
        conv = _combine_call(
            acc.reshape(NC, 2, NB, WCOL), a_s, a_d, h, b.reshape(1, D))
        convs.append(conv)
        feat = jnp.pad(conv, ((0, NB - N), (0, 0)))

    h1, stats = _mlp1_call(x, convs[0], convs[1], convs[2],
                           mlp_W1.reshape(4, D, D), mlp_b1.reshape(1, D))
    return _mlp2_call(h1, stats, mlp_gamma.reshape(1, D),
                      mlp_beta.reshape(1, D), mlp_W2, mlp_b2.reshape(1, D))


# speedup vs baseline: 2.3050x; 1.1911x over previous
import functools

import jax
import jax.numpy as jnp
from jax import lax
from jax.experimental import pallas as pl
from jax.experimental.pallas import tpu as pltpu
from jax.experimental.pallas import tpu_sc as plsc

N = 10000
E = 320000
D = 128
HALF = D // 2
WCOL = 64
DST = 80
NB = 10240
NC = 2
NS = 16
L = 16
CHW = 128
NCH = 160
EPT = NCH * CHW
ESLOTS = NS * EPT
STRIPE = 2 * NB // NS

_f32 = jnp.float32
_i32 = jnp.int32


def _build_body(x_ref, w_ref, avs_ref, avd_ref, g_ref, h_ref, as_ref, ad_ref):
    h = jnp.dot(x_ref[...], w_ref[...], preferred_element_type=_f32)
    a_s = jnp.dot(h, avs_ref[...], preferred_element_type=_f32)
    a_d = jnp.dot(h, avd_ref[...], preferred_element_type=_f32)
    e1 = jnp.exp(a_s)
    e2 = jnp.exp(0.2 * a_s)
    lo = h[:, :HALF]
    hi = h[:, HALF:]
    g_ref[...] = jnp.stack([e1 * lo, e2 * lo, e1 * hi, e2 * hi], axis=0)
    h_ref[...] = h
    as_ref[...] = a_s
    ad_ref[...] = a_d


def _build_call(x_pad, W, av_s, av_d):
    BN = 1280
    grid = NB // BN
    return pl.pallas_call(
        _build_body,
        grid=(grid,),
        in_specs=[
            pl.BlockSpec((BN, D), lambda i: (i, 0)),
            pl.BlockSpec((D, D), lambda i: (0, 0)),
            pl.BlockSpec((D, 1), lambda i: (0, 0)),
            pl.BlockSpec((D, 1), lambda i: (0, 0)),
        ],
        out_specs=[
            pl.BlockSpec((4, BN, WCOL), lambda i: (0, i, 0)),
            pl.BlockSpec((BN, D), lambda i: (i, 0)),
            pl.BlockSpec((BN, 1), lambda i: (i, 0)),
            pl.BlockSpec((BN, 1), lambda i: (i, 0)),
        ],
        out_shape=[
            jax.ShapeDtypeStruct((4, NB, WCOL), _f32),
            jax.ShapeDtypeStruct((NB, D), _f32),
            jax.ShapeDtypeStruct((NB, 1), _f32),
            jax.ShapeDtypeStruct((NB, 1), _f32),
        ],
    )(x_pad, W, av_s, av_d)


def _sign_body(as_hbm, ad_hbm, src_hbm, dst_hbm, gsrc_hbm, gdst_hbm, den_hbm,
               asv, adv, srcv, dstv, denv, idv, den_sh):
    c = lax.axis_index("c")
    s = lax.axis_index("s")

    pltpu.sync_copy(as_hbm, asv)
    pltpu.sync_copy(ad_hbm, adv)
    half = NCH // NC
    pltpu.sync_copy(src_hbm.at[s, pl.ds(c * half, half)], srcv)
    pltpu.sync_copy(dst_hbm.at[s, pl.ds(c * half, half)], dstv)

    zero16 = jnp.zeros((L,), _f32)
    lane = jnp.arange(L, dtype=_i32)

    def zden(r, carry):
        for q in range(128 // L):
            denv[r, pl.ds(q * L, L)] = zero16
        return carry

    lax.fori_loop(0, DST, zden, 0)
    for q in range(DST // L):
        idv[pl.ds(q * L, L)] = lane + (q * L)
    dstripe = DST // NS
    pltpu.sync_copy(denv.at[pl.ds(s * dstripe, dstripe)],
                    den_sh.at[pl.ds(s * dstripe, dstripe)])
    plsc.subcore_barrier()

    nbvec = jnp.full((L,), NB, _i32)
    zvec = jnp.zeros((L,), _i32)
    c1vec = jnp.full((L,), 2 * NB, _i32)
    m127 = jnp.full((L,), 127, _i32)

    def sgn(j, carry):
        for i in range(CHW // L):
            s16 = srcv[j, pl.ds(i * L, L)]
            d16 = dstv[j, pl.ds(i * L, L)]
            z = plsc.load_gather(asv, [s16]) + plsc.load_gather(adv, [d16])
            neg = z < 0.0
            w16 = jnp.exp(jnp.where(neg, 0.2 * z, z))
            plsc.addupdate_scatter(
                denv, [lax.shift_right_logical(d16, 7), d16 & m127], w16)
            offs = jnp.where(neg, nbvec, zvec)
            srcv[j, pl.ds(i * L, L)] = s16 + offs
            dstv[j, pl.ds(i * L, L)] = d16 + offs
        return carry

    lax.fori_loop(0, half, sgn, 0)
    pltpu.sync_copy(denv, den_sh.at[idv], add=True)
    plsc.subcore_barrier()
    pltpu.sync_copy(den_sh.at[pl.ds(s * dstripe, dstripe)],
                    den_hbm.at[c, pl.ds(s * dstripe, dstripe)])
    pltpu.sync_copy(srcv, gsrc_hbm.at[0, s, pl.ds(c * half, half)])
    pltpu.sync_copy(dstv, gdst_hbm.at[s, pl.ds(c * half, half)])

    def sft(j, carry):
        for i in range(CHW // L):
            srcv[j, pl.ds(i * L, L)] += c1vec
        return carry

    lax.fori_loop(0, half, sft, 0)
    pltpu.sync_copy(srcv, gsrc_hbm.at[1, s, pl.ds(c * half, half)])


def _sign_call():
  return pl.kernel(
    _sign_body,
    out_type=[
        jax.ShapeDtypeStruct((NC, NS, NCH, CHW), _i32),
        jax.ShapeDtypeStruct((NS, NCH, CHW), _i32),
        jax.ShapeDtypeStruct((NC, DST, 128), _f32),
    ],
    mesh=plsc.VectorSubcoreMesh(core_axis_name="c", subcore_axis_name="s"),
    compiler_params=pltpu.CompilerParams(needs_layout_passes=False,
                                         use_tc_tiling_on_sc=False),
    scratch_types=[
        pltpu.VMEM((NB,), _f32),
        pltpu.VMEM((NB,), _f32),
        pltpu.VMEM((NCH // NC, CHW), _i32),
        pltpu.VMEM((NCH // NC, CHW), _i32),
        pltpu.VMEM((DST, 128), _f32),
        pltpu.VMEM((DST,), _i32),
        pltpu.VMEM_SHARED((DST, 128), _f32),
    ],
  )


BLK = 4
RING = 4
NBLK = NCH // BLK


def _gs_body(g_hbm, gsrc_hbm, gdst_hbm, out_hbm, sidx, didx, rows, acc,
             semg0, semg1, sems0, sems1, si0, si1, si2, si3):
    c = lax.axis_index("c")
    s = lax.axis_index("s")
    semg = (semg0, semg1)
    sems = (sems0, sems1)
    semi = (si0, si1, si2, si3)

    zero16 = jnp.zeros((L,), _f32)

    def zrow(r, carry):
        for q in range(WCOL // L):
            rows[0, r, pl.ds(q * L, L)] = zero16
        return carry

    lax.fori_loop(0, CHW, zrow, 0)
    base = s * STRIPE
    for k in range(STRIPE // CHW):
        pltpu.sync_copy(rows.at[0], acc.at[pl.ds(base + k * CHW, CHW)])
    plsc.subcore_barrier()

    def stage(k, sl):
        pltpu.async_copy(gsrc_hbm.at[c, s, pl.ds(k * BLK, BLK)],
                         sidx.at[sl], semi[sl])
        pltpu.async_copy(gdst_hbm.at[s, pl.ds(k * BLK, BLK)],
                         didx.at[sl], semi[sl])

    def stage_wait(k, sl):
        pltpu.make_async_copy(gsrc_hbm.at[c, s, pl.ds(k * BLK, BLK)],
                              sidx.at[sl], semi[sl]).wait()
        pltpu.make_async_copy(gdst_hbm.at[s, pl.ds(k * BLK, BLK)],
                              didx.at[sl], semi[sl]).wait()

    def drain_scatter(b):
        pltpu.make_async_copy(rows.at[b], acc.at[didx.at[0, 0]],
                              sems[b]).wait()

    def drain_gather(b):
        pltpu.make_async_copy(g_hbm.at[sidx.at[0, 0]], rows.at[b],
                              semg[b]).wait()

    def block(k, sl, psl, first):
        stage_wait(k, sl)
        for m in range(BLK):
            b = m % 2
            nb = 1 - b
            if not (first and m < 2):
                drain_scatter(b)
            pltpu.async_copy(g_hbm.at[sidx.at[sl, m]], rows.at[b], semg[b])
            if not (first and m == 0):
                drain_gather(nb)
                if m > 0:
                    pltpu.async_copy(rows.at[nb], acc.at[didx.at[sl, m - 1]],
                                     sems[nb], add=True)
                else:
                    pltpu.async_copy(rows.at[nb],
                                     acc.at[didx.at[psl, BLK - 1]],
                                     sems[nb], add=True)

    for kk in range(RING):
        stage(kk, kk)
    block(0, 0, None, True)
    block(1, 1, 0, False)
    stage(4, 0)
    block(2, 2, 1, False)
    stage(5, 1)
    block(3, 3, 2, False)

    def blocks(g, carry):
        for b4 in range(RING):
            k = RING * g + b4

            @pl.when(k + 2 < NBLK)
            def _():
                stage(k + 2, (b4 + 2) % RING)

            block(k, b4, (b4 - 1) % RING, False)
        return carry

    lax.fori_loop(1, NBLK // RING, blocks, 0)

    drain_gather(1)
    pltpu.async_copy(rows.at[1], acc.at[didx.at[RING - 1, BLK - 1]],
                     sems[1], add=True)
    drain_scatter(0)
    drain_scatter(1)
    plsc.subcore_barrier()

    pltpu.sync_copy(acc.at[pl.ds(base, STRIPE)],
                    out_hbm.at[c, pl.ds(base, STRIPE)])


def _gs_call():
  return pl.kernel(
    _gs_body,
    out_type=jax.ShapeDtypeStruct((NC, 2 * NB, WCOL), _f32),
    mesh=plsc.VectorSubcoreMesh(core_axis_name="c", subcore_axis_name="s"),
    compiler_params=pltpu.CompilerParams(needs_layout_passes=False,
                                         use_tc_tiling_on_sc=False),
    scratch_types=[
        pltpu.VMEM((RING, BLK, CHW), _i32),
        pltpu.VMEM((RING, BLK, CHW), _i32),
        pltpu.VMEM((2, CHW, WCOL), _f32),
        pltpu.VMEM_SHARED((2 * NB, WCOL), _f32),
        pltpu.SemaphoreType.DMA,
        pltpu.SemaphoreType.DMA,
        pltpu.SemaphoreType.DMA,
        pltpu.SemaphoreType.DMA,
        pltpu.SemaphoreType.DMA,
        pltpu.SemaphoreType.DMA,
        pltpu.SemaphoreType.DMA,
        pltpu.SemaphoreType.DMA,
    ],
  )


def _sc_edge_pass(G, a_s, a_d, srcp, dstp):
    gsrc, gdst, den = _sign_call()(a_s, a_d, srcp, dstp)
    return _gs_call()(G, gsrc, gdst), den


def _combine_body(acc_ref, den_ref, as_ref, ad_ref, h_ref, b_ref, out_ref):
    a_s = as_ref[...]
    a_d = ad_ref[...]
    h = h_ref[...]
    e1 = jnp.exp(a_d)
    e2 = jnp.exp(0.2 * a_d)
    lo = e1 * acc_ref[0, 0] + e2 * acc_ref[0, 1]
    hi = e1 * acc_ref[1, 0] + e2 * acc_ref[1, 1]
    z = a_s + a_d
    ws = jnp.exp(jnp.where(z >= 0.0, z, 0.2 * z))
    num = jnp.concatenate([lo, hi], axis=1) + ws * h
    den = den_ref[0] + den_ref[1] + ws
    out_ref[...] = num / den + b_ref[...]


def _combine_call(acc4, den, a_s, a_d, h, bias):
    BN = 1000
    grid = N // BN
    return pl.pallas_call(
        _combine_body,
        grid=(grid,),
        in_specs=[
            pl.BlockSpec((2, 2, BN, WCOL), lambda i: (0, 0, i, 0)),
            pl.BlockSpec((2, BN, 1), lambda i: (0, i, 0)),
            pl.BlockSpec((BN, 1), lambda i: (i, 0)),
            pl.BlockSpec((BN, 1), lambda i: (i, 0)),
            pl.BlockSpec((BN, D), lambda i: (i, 0)),
            pl.BlockSpec((1, D), lambda i: (0, 0)),
        ],
        out_specs=pl.BlockSpec((BN, D), lambda i: (i, 0)),
        out_shape=jax.ShapeDtypeStruct((N, D), _f32),
    )(acc4, den, a_s, a_d, h, bias)


def _mlp1_body(x_ref, c0_ref, c1_ref, c2_ref, w1_ref, b1_ref, h1_ref, st_ref):
    h1 = (jnp.dot(x_ref[...], w1_ref[0], preferred_element_type=_f32)
          + jnp.dot(c0_ref[...], w1_ref[1], preferred_element_type=_f32)
          + jnp.dot(c1_ref[...], w1_ref[2], preferred_element_type=_f32)
          + jnp.dot(c2_ref[...], w1_ref[3], preferred_element_type=_f32)
          + b1_ref[...])
    h1_ref[...] = h1
    part = jnp.concatenate([jnp.sum(h1, axis=0, keepdims=True),
                            jnp.sum(h1 * h1, axis=0, keepdims=True)], axis=0)

    @pl.when(pl.program_id(0) == 0)
    def _():
        st_ref[...] = jnp.zeros_like(st_ref)

    st_ref[...] += part


def _mlp1_call(x, c0, c1, c2, w1_4, b1):
    BN = 1000
    grid = N // BN
    return pl.pallas_call(
        _mlp1_body,
        grid=(grid,),
        in_specs=[
            pl.BlockSpec((BN, D), lambda i: (i, 0)),
            pl.BlockSpec((BN, D), lambda i: (i, 0)),
            pl.BlockSpec((BN, D), lambda i: (i, 0)),
            pl.BlockSpec((BN, D), lambda i: (i, 0)),
            pl.BlockSpec((4, D, D), lambda i: (0, 0, 0)),
            pl.BlockSpec((1, D), lambda i: (0, 0)),
        ],
        out_specs=[
            pl.BlockSpec((BN, D), lambda i: (i, 0)),
            pl.BlockSpec((2, D), lambda i: (0, 0)),
        ],
        out_shape=[
            jax.ShapeDtypeStruct((N, D), _f32),
            jax.ShapeDtypeStruct((2, D), _f32),
        ],
    )(x, c0, c1, c2, w1_4, b1)


def _mlp2_body(h1_ref, st_ref, g_ref, be_ref, w2_ref, b2_ref, out_ref):
    mean = st_ref[0:1] * (1.0 / N)
    var = st_ref[1:2] * (1.0 / N) - mean * mean
    xn = (h1_ref[...] - mean) * lax.rsqrt(var + 1e-5) * g_ref[...] + be_ref[...]
    xn = jnp.maximum(xn, 0.0)
    out_ref[...] = jnp.dot(xn, w2_ref[...], preferred_element_type=_f32) + b2_ref[...]


def _mlp2_call(h1, stats, gamma, beta, w2, b2):
    BN = 1000
    grid = N // BN
    return pl.pallas_call(
        _mlp2_body,
        grid=(grid,),
        in_specs=[
            pl.BlockSpec((BN, D), lambda i: (i, 0)),
            pl.BlockSpec((2, D), lambda i: (0, 0)),
            pl.BlockSpec((1, D), lambda i: (0, 0)),
            pl.BlockSpec((1, D), lambda i: (0, 0)),
            pl.BlockSpec((D, D), lambda i: (0, 0)),
            pl.BlockSpec((1, D), lambda i: (0, 0)),
        ],
        out_specs=pl.BlockSpec((BN, D), lambda i: (i, 0)),
        out_shape=jax.ShapeDtypeStruct((N, D), _f32),
    )(h1, stats, gamma, beta, w2, b2)


def kernel(x, edge_index, t, batch_ids,
           W0, att_src0, att_dst0, bias0,
           W1, att_src1, att_dst1, bias1,
           W2, att_src2, att_dst2, bias2,
           mlp_W1, mlp_b1, mlp_gamma, mlp_beta, mlp_W2, mlp_b2):
    del t, batch_ids
    pad_e = ESLOTS - E
    srcp = jnp.concatenate(
        [edge_index[0], jnp.full((pad_e,), N, _i32)]).reshape(NS, NCH, CHW)
    dstp = jnp.concatenate(
        [edge_index[1], jnp.full((pad_e,), N, _i32)]).reshape(NS, NCH, CHW)

    feat = jnp.pad(x, ((0, NB - N), (0, 0)))
    convs = []
    params = [(W0, att_src0, att_dst0, bias0),
              (W1, att_src1, att_dst1, bias1),
              (W2, att_src2, att_dst2, bias2)]
    for (W, a_s_w, a_d_w, b) in params:
        G4, h, a_s, a_d = _build_call(
            feat, W, a_s_w.reshape(D, 1), a_d_w.reshape(D, 1))
        acc, den = _sc_edge_pass(
            G4.reshape(4 * NB, WCOL), a_s.reshape(NB), a_d.reshape(NB),
            srcp, dstp)
        conv = _combine_call(
            acc.reshape(NC, 2, NB, WCOL), den.reshape(NC, NB, 1)[:, :N],
            a_s, a_d, h, b.reshape(1, D))
        convs.append(conv)
        feat = jnp.pad(conv, ((0, NB - N), (0, 0)))

    h1, stats = _mlp1_call(x, convs[0], convs[1], convs[2],
                           mlp_W1.reshape(4, D, D), mlp_b1.reshape(1, D))
    return _mlp2_call(h1, stats, mlp_gamma.reshape(1, D),
                      mlp_beta.reshape(1, D), mlp_W2, mlp_b2.reshape(1, D))
